# Initial kernel scaffold; baseline (speedup 1.0000x reference)
import jax
import jax.numpy as jnp
from jax.experimental import pallas as pl


def kernel(x, edge_index, batch, W_fc, b_fc, W_nbr, b_nbr, W_root, ln_g, ln_b):
    raise NotImplementedError("write your pallas kernel here")


# speedup vs baseline: 4.8610x; 4.8610x over previous
import jax
import jax.numpy as jnp
from jax import lax
from jax.experimental import pallas as pl
from jax.experimental.pallas import tpu as pltpu
from jax.experimental.pallas import tpu_sc as plsc

N = 10000
E = 320000
D_IN = 128
H = 64
L = 3
EPS = 1e-5

NC = 2
NS = 16
NW = NC * NS
LANES = 128
ROWS_W = 80
E_PAD = NW * ROWS_W * LANES
NPAD = 10240
RPT = NPAD // NS

_MESH = plsc.VectorSubcoreMesh(
    core_axis_name="c", subcore_axis_name="s", num_cores=NC, num_subcores=NS)


def _sc_agg_body(h_hbm, src_hbm, dst_hbm, zeros_hbm, out_hbm,
                 src_v, dst_v, rows_v, agg_sh, sem):
  cid = lax.axis_index("c")
  sid = lax.axis_index("s")
  wid = sid * NC + cid
  r0 = sid * RPT
  pltpu.sync_copy(zeros_hbm.at[pl.ds(r0, RPT)], agg_sh.at[pl.ds(r0, RPT)])
  pltpu.sync_copy(src_hbm.at[pl.ds(wid * ROWS_W, ROWS_W)], src_v)
  pltpu.sync_copy(dst_hbm.at[pl.ds(wid * ROWS_W, ROWS_W)], dst_v)
  plsc.subcore_barrier()

  def body(j, carry):
    pltpu.async_copy(h_hbm.at[src_v.at[j]], rows_v, sem).wait()
    pltpu.sync_copy(rows_v, agg_sh.at[dst_v.at[j]], add=True)
    return carry

  lax.fori_loop(0, ROWS_W, body, 0)
  plsc.subcore_barrier()
  pltpu.sync_copy(agg_sh.at[pl.ds(r0, RPT)], out_hbm.at[cid, pl.ds(r0, RPT)])


_sc_agg = pl.kernel(
    _sc_agg_body,
    out_type=jax.ShapeDtypeStruct((NC, NPAD, H), jnp.float32),
    mesh=_MESH,
    compiler_params=pltpu.CompilerParams(use_tc_tiling_on_sc=False),
    scratch_types=[
        pltpu.VMEM((ROWS_W, LANES), jnp.int32),
        pltpu.VMEM((ROWS_W, LANES), jnp.int32),
        pltpu.VMEM((LANES, H), jnp.float32),
        pltpu.VMEM_SHARED((NPAD, H), jnp.float32),
        pltpu.SemaphoreType.DMA,
    ],
)


def _sc_deg_body(dst_hbm, ones_hbm, zeros_hbm, out_hbm,
                 dst_v, ones_v, deg_sh, sem):
  cid = lax.axis_index("c")
  sid = lax.axis_index("s")
  wid = sid * NC + cid
  r0 = sid * RPT
  pltpu.sync_copy(zeros_hbm.at[pl.ds(r0, RPT)], deg_sh.at[pl.ds(r0, RPT)])
  pltpu.sync_copy(ones_hbm, ones_v)
  pltpu.sync_copy(dst_hbm.at[pl.ds(wid * ROWS_W, ROWS_W)], dst_v)
  plsc.subcore_barrier()

  def body(j, carry):
    pltpu.sync_copy(ones_v, deg_sh.at[dst_v.at[j]], add=True)
    return carry

  lax.fori_loop(0, ROWS_W, body, 0)
  plsc.subcore_barrier()
  pltpu.sync_copy(deg_sh.at[pl.ds(r0, RPT)], out_hbm.at[cid, pl.ds(r0, RPT)])


_sc_deg = pl.kernel(
    _sc_deg_body,
    out_type=jax.ShapeDtypeStruct((NC, NPAD, 16), jnp.float32),
    mesh=_MESH,
    compiler_params=pltpu.CompilerParams(use_tc_tiling_on_sc=False),
    scratch_types=[
        pltpu.VMEM((ROWS_W, LANES), jnp.int32),
        pltpu.VMEM((LANES, 16), jnp.float32),
        pltpu.VMEM_SHARED((NPAD, 16), jnp.float32),
        pltpu.SemaphoreType.DMA,
    ],
)

BN = 1000


def _fc_body(x_ref, w_ref, b_ref, o_ref):
  o_ref[...] = jnp.maximum(
      jnp.dot(x_ref[...], w_ref[...], preferred_element_type=jnp.float32)
      + b_ref[...], 0.0)


_fc = pl.pallas_call(
    _fc_body,
    grid=(N // BN,),
    in_specs=[
        pl.BlockSpec((BN, D_IN), lambda i: (i, 0)),
        pl.BlockSpec((D_IN, H), lambda i: (0, 0)),
        pl.BlockSpec((1, H), lambda i: (0, 0)),
    ],
    out_specs=pl.BlockSpec((BN, H), lambda i: (i, 0)),
    out_shape=jax.ShapeDtypeStruct((N, H), jnp.float32),
)


def _make_layer(with_ln):
  def body(parts_ref, degp_ref, h_ref, wn_ref, bn_ref, wr_ref, g_ref, be_ref,
           o_ref):
    s = parts_ref[0] + parts_ref[1]
    deg = jnp.maximum(degp_ref[0, :, 0:1] + degp_ref[1, :, 0:1], 1.0)
    hn = (jnp.dot(s / deg, wn_ref[...], preferred_element_type=jnp.float32)
          + bn_ref[...]
          + jnp.dot(h_ref[...], wr_ref[...], preferred_element_type=jnp.float32))
    if with_ln:
      mu = jnp.mean(hn, axis=-1, keepdims=True)
      var = jnp.mean((hn - mu) ** 2, axis=-1, keepdims=True)
      hn = g_ref[...] * (hn - mu) * lax.rsqrt(var + EPS) + be_ref[...]
      hn = jnp.maximum(hn, 0.0)
    o_ref[...] = hn

  return pl.pallas_call(
      body,
      grid=(N // BN,),
      in_specs=[
          pl.BlockSpec((NC, BN, H), lambda i: (0, i, 0)),
          pl.BlockSpec((NC, BN, 16), lambda i: (0, i, 0)),
          pl.BlockSpec((BN, H), lambda i: (i, 0)),
          pl.BlockSpec((H, H), lambda i: (0, 0)),
          pl.BlockSpec((1, H), lambda i: (0, 0)),
          pl.BlockSpec((H, H), lambda i: (0, 0)),
          pl.BlockSpec((1, H), lambda i: (0, 0)),
          pl.BlockSpec((1, H), lambda i: (0, 0)),
      ],
      out_specs=pl.BlockSpec((BN, H), lambda i: (i, 0)),
      out_shape=jax.ShapeDtypeStruct((N, H), jnp.float32),
  )


_layer_ln = _make_layer(True)
_layer_last = _make_layer(False)


def kernel(x, edge_index, batch, W_fc, b_fc, W_nbr, b_nbr, W_root, ln_g, ln_b):
  del batch
  src = edge_index[0]
  dst = edge_index[1]
  pad = E_PAD - E
  src_p = jnp.concatenate(
      [src, jnp.zeros((pad,), jnp.int32)]).reshape(NW * ROWS_W, LANES)
  dst_p = jnp.concatenate(
      [dst, jnp.full((pad,), NPAD - 1, jnp.int32)]).reshape(NW * ROWS_W, LANES)
  zeros_h = jnp.zeros((NPAD, H), jnp.float32)
  zeros_16 = jnp.zeros((NPAD, 16), jnp.float32)
  ones_16 = jnp.ones((LANES, 16), jnp.float32)

  h = _fc(x, W_fc, b_fc.reshape(1, H))
  degp = _sc_deg(dst_p, ones_16, zeros_16)
  for l in range(L):
    parts = _sc_agg(h, src_p, dst_p, zeros_h)
    f = _layer_ln if l < L - 1 else _layer_last
    g = ln_g[l] if l < L - 1 else ln_g[0]
    b = ln_b[l] if l < L - 1 else ln_b[0]
    h = f(parts, degp, h, W_nbr[l], b_nbr[l].reshape(1, H), W_root[l],
          g.reshape(1, H), b.reshape(1, H))
  return h


# speedup vs baseline: 13.3130x; 2.7388x over previous
import jax
import jax.numpy as jnp
from jax import lax
from jax.experimental import pallas as pl
from jax.experimental.pallas import tpu as pltpu
from jax.experimental.pallas import tpu_sc as plsc

N = 10000
E = 320000
D_IN = 128
H = 64
L = 3
EPS = 1e-5

NC = 2
NS = 16
NW = NC * NS
E_W = E // NW
GB = 200
PH = E_W // GB
NPAD = 10240
RPT = NPAD // NS
HPT = N // NS

_MESH = plsc.VectorSubcoreMesh(
    core_axis_name="c", subcore_axis_name="s", num_cores=NC, num_subcores=NS)


def _make_sc_agg(with_deg):
  def body(h_hbm, e_hbm, zeros_hbm, *rest):
    if with_deg:
      (out_hbm, deg_hbm, ones_v, ebuf, rows_v, h_sh, agg_sh, deg_sh,
       isem, gsem, ssem, dsem) = rest
    else:
      (out_hbm, ebuf, rows_v, h_sh, agg_sh, isem, gsem, ssem) = rest
    cid = lax.axis_index("c")
    sid = lax.axis_index("s")
    wid = sid * NC + cid
    r0 = sid * RPT
    h0 = sid * HPT
    e0 = wid * E_W

    def idx_start(ph, q):
      off = e0 + ph * GB
      pltpu.async_copy(e_hbm.at[0, pl.ds(off, GB)], ebuf.at[q, 0], isem.at[q])
      pltpu.async_copy(e_hbm.at[1, pl.ds(off, GB)], ebuf.at[q, 1], isem.at[q])

    def idx_wait(ph, q):
      off = e0 + ph * GB
      pltpu.make_async_copy(e_hbm.at[0, pl.ds(off, GB)], ebuf.at[q, 0],
                            isem.at[q]).wait()
      pltpu.make_async_copy(e_hbm.at[1, pl.ds(off, GB)], ebuf.at[q, 1],
                            isem.at[q]).wait()

    def drain(q, r):
      pltpu.make_async_copy(rows_v.at[r], agg_sh.at[ebuf.at[q, 1]],
                            ssem.at[r]).wait()
      if with_deg:
        pltpu.make_async_copy(ones_v, deg_sh.at[ebuf.at[q, 1]],
                              dsem.at[r]).wait()

    def gather_scatter(q, r):
      pltpu.async_copy(h_sh.at[ebuf.at[q, 0]], rows_v.at[r], gsem.at[r]).wait()
      pltpu.async_copy(rows_v.at[r], agg_sh.at[ebuf.at[q, 1]], ssem.at[r],
                       add=True)
      if with_deg:
        pltpu.async_copy(ones_v, deg_sh.at[ebuf.at[q, 1]], dsem.at[r],
                         add=True)

    pltpu.sync_copy(h_hbm.at[pl.ds(h0, HPT)], h_sh.at[pl.ds(h0, HPT)])
    pltpu.sync_copy(zeros_hbm.at[pl.ds(r0, RPT)], agg_sh.at[pl.ds(r0, RPT)])
    if with_deg:
      pltpu.sync_copy(zeros_hbm.at[pl.ds(r0, RPT), pl.ds(0, 16)],
                      deg_sh.at[pl.ds(r0, RPT)])
      pltpu.sync_copy(ones_hbm_ref(rest), ones_v)
    for q in range(4):
      idx_start(q, q)
    plsc.subcore_barrier()

    for q in range(4):
      idx_wait(q, q)
      if q >= 2:
        drain(q - 2, q % 2)
        idx_start(q + 2, q - 2)
      gather_scatter(q, q % 2)

    def g4body(g4, carry):
      for q in range(4):
        ph = g4 * 4 + q
        drain((q + 2) % 4, q % 2)
        pltpu.async_copy(e_hbm.at[0, pl.ds(e0 + (ph + 2) * GB, GB)],
                         ebuf.at[(q + 2) % 4, 0], isem.at[(q + 2) % 4])
        pltpu.async_copy(e_hbm.at[1, pl.ds(e0 + (ph + 2) * GB, GB)],
                         ebuf.at[(q + 2) % 4, 1], isem.at[(q + 2) % 4])
        idx_wait(ph, q)
        gather_scatter(q, q % 2)
      return carry

    lax.fori_loop(1, PH // 4, g4body, 0)

    for q in range(2):
      ph = PH - 2 + q
      drain((q + 2) % 4, q % 2)
      idx_wait(ph, q)
      gather_scatter(q, q % 2)
    for q in range(2):
      drain(q, q % 2)

    plsc.subcore_barrier()
    pltpu.sync_copy(agg_sh.at[pl.ds(r0, RPT)],
                    out_hbm.at[pl.ds(r0, RPT), pl.ds(cid * H, H)])
    if with_deg:
      pltpu.sync_copy(deg_sh.at[pl.ds(r0, RPT)],
                      deg_hbm.at[pl.ds(r0, RPT), pl.ds(cid * 16, 16)])

  def ones_hbm_ref(rest):
    raise AssertionError

  if with_deg:
    def body_deg(h_hbm, e_hbm, zeros_hbm, ones_hbm, out_hbm, deg_hbm,
                 ones_v, ebuf, rows_v, h_sh, agg_sh, deg_sh,
                 isem, gsem, ssem, dsem):
      nonlocal ones_hbm_ref
      ones_hbm_ref = lambda rest: ones_hbm
      return body(h_hbm, e_hbm, zeros_hbm, out_hbm, deg_hbm, ones_v, ebuf,
                  rows_v, h_sh, agg_sh, deg_sh, isem, gsem, ssem, dsem)

    out_type = (jax.ShapeDtypeStruct((NPAD, NC * H), jnp.float32),
                jax.ShapeDtypeStruct((NPAD, NC * 16), jnp.float32))
    scratch = [
        pltpu.VMEM((GB, 16), jnp.float32),
        pltpu.VMEM((4, 2, GB), jnp.int32),
        pltpu.VMEM((2, GB, H), jnp.float32),
        pltpu.VMEM_SHARED((N, H), jnp.float32),
        pltpu.VMEM_SHARED((NPAD, H), jnp.float32),
        pltpu.VMEM_SHARED((NPAD, 16), jnp.float32),
        pltpu.SemaphoreType.DMA((4,)),
        pltpu.SemaphoreType.DMA((2,)),
        pltpu.SemaphoreType.DMA((2,)),
        pltpu.SemaphoreType.DMA((2,)),
    ]
    fn = body_deg
  else:
    out_type = jax.ShapeDtypeStruct((NPAD, NC * H), jnp.float32)
    scratch = [
        pltpu.VMEM((4, 2, GB), jnp.int32),
        pltpu.VMEM((2, GB, H), jnp.float32),
        pltpu.VMEM_SHARED((N, H), jnp.float32),
        pltpu.VMEM_SHARED((NPAD, H), jnp.float32),
        pltpu.SemaphoreType.DMA((4,)),
        pltpu.SemaphoreType.DMA((2,)),
        pltpu.SemaphoreType.DMA((2,)),
    ]
    fn = body

  return pl.kernel(
      fn,
      out_type=out_type,
      mesh=_MESH,
      compiler_params=pltpu.CompilerParams(use_tc_tiling_on_sc=False),
      scratch_types=scratch,
  )


_sc_agg_deg = _make_sc_agg(True)
_sc_agg = _make_sc_agg(False)

BN = 1000


def _fc_body(x_ref, w_ref, b_ref, o_ref):
  o_ref[...] = jnp.maximum(
      jnp.dot(x_ref[...], w_ref[...], preferred_element_type=jnp.float32)
      + b_ref[...], 0.0)


_fc = pl.pallas_call(
    _fc_body,
    grid=(N // BN,),
    in_specs=[
        pl.BlockSpec((BN, D_IN), lambda i: (i, 0)),
        pl.BlockSpec((D_IN, H), lambda i: (0, 0)),
        pl.BlockSpec((1, H), lambda i: (0, 0)),
    ],
    out_specs=pl.BlockSpec((BN, H), lambda i: (i, 0)),
    out_shape=jax.ShapeDtypeStruct((N, H), jnp.float32),
)


def _make_layer(with_ln):
  def body(parts_ref, degp_ref, h_ref, wn_ref, bn_ref, wr_ref, g_ref, be_ref,
           o_ref):
    p = parts_ref[...]
    s = p[:, :H] + p[:, H:]
    d = degp_ref[...]
    deg = jnp.maximum(d[:, 0:1] + d[:, 16:17], 1.0)
    hn = (jnp.dot(s / deg, wn_ref[...], preferred_element_type=jnp.float32)
          + bn_ref[...]
          + jnp.dot(h_ref[...], wr_ref[...], preferred_element_type=jnp.float32))
    if with_ln:
      mu = jnp.mean(hn, axis=-1, keepdims=True)
      var = jnp.mean((hn - mu) ** 2, axis=-1, keepdims=True)
      hn = g_ref[...] * (hn - mu) * lax.rsqrt(var + EPS) + be_ref[...]
      hn = jnp.maximum(hn, 0.0)
    o_ref[...] = hn

  return pl.pallas_call(
      body,
      grid=(N // BN,),
      in_specs=[
          pl.BlockSpec((BN, NC * H), lambda i: (i, 0)),
          pl.BlockSpec((BN, NC * 16), lambda i: (i, 0)),
          pl.BlockSpec((BN, H), lambda i: (i, 0)),
          pl.BlockSpec((H, H), lambda i: (0, 0)),
          pl.BlockSpec((1, H), lambda i: (0, 0)),
          pl.BlockSpec((H, H), lambda i: (0, 0)),
          pl.BlockSpec((1, H), lambda i: (0, 0)),
          pl.BlockSpec((1, H), lambda i: (0, 0)),
      ],
      out_specs=pl.BlockSpec((BN, H), lambda i: (i, 0)),
      out_shape=jax.ShapeDtypeStruct((N, H), jnp.float32),
  )


_layer_ln = _make_layer(True)
_layer_last = _make_layer(False)


def kernel(x, edge_index, batch, W_fc, b_fc, W_nbr, b_nbr, W_root, ln_g, ln_b):
  del batch
  zeros_h = jnp.zeros((NPAD, H), jnp.float32)
  ones_16 = jnp.ones((GB, 16), jnp.float32)

  h = _fc(x, W_fc, b_fc.reshape(1, H))
  degp = None
  for l in range(L):
    if l == 0:
      parts, degp = _sc_agg_deg(h, edge_index, zeros_h, ones_16)
    else:
      parts = _sc_agg(h, edge_index, zeros_h)
    f = _layer_ln if l < L - 1 else _layer_last
    g = ln_g[l] if l < L - 1 else ln_g[0]
    b = ln_b[l] if l < L - 1 else ln_b[0]
    h = f(parts, degp, h, W_nbr[l], b_nbr[l].reshape(1, H), W_root[l],
          g.reshape(1, H), b.reshape(1, H))
  return h


# speedup vs baseline: 16.1670x; 1.2144x over previous
import jax
import jax.numpy as jnp
from jax import lax
from jax.experimental import pallas as pl
from jax.experimental.pallas import tpu as pltpu
from jax.experimental.pallas import tpu_sc as plsc

N = 10000
E = 320000
D_IN = 128
H = 64
L = 3
EPS = 1e-5

NC = 2
NS = 16
NW = NC * NS
E_W = E // NW
GB = 200
PH = E_W // GB
NPAD = 10240
RPT = NPAD // NS
HPT = N // NS

_MESH = plsc.VectorSubcoreMesh(
    core_axis_name="c", subcore_axis_name="s", num_cores=NC, num_subcores=NS)


def _make_sc_agg(with_deg):
  def body(h_hbm, e_hbm, zeros_hbm, *rest):
    if with_deg:
      (out_hbm, deg_hbm, ones_v, ebuf, rows_v, h_sh, agg_sh, deg_sh,
       isem, gsem, ssem, dsem) = rest
    else:
      (out_hbm, ebuf, rows_v, h_sh, agg_sh, isem, gsem, ssem) = rest
    cid = lax.axis_index("c")
    sid = lax.axis_index("s")
    wid = sid * NC + cid
    r0 = sid * RPT
    h0 = sid * HPT
    e0 = wid * E_W

    def idx_start(ph, q):
      off = e0 + ph * GB
      pltpu.async_copy(e_hbm.at[0, pl.ds(off, GB)], ebuf.at[q, 0], isem.at[q])
      pltpu.async_copy(e_hbm.at[1, pl.ds(off, GB)], ebuf.at[q, 1], isem.at[q])

    def idx_wait(ph, q):
      off = e0 + ph * GB
      pltpu.make_async_copy(e_hbm.at[0, pl.ds(off, GB)], ebuf.at[q, 0],
                            isem.at[q]).wait()
      pltpu.make_async_copy(e_hbm.at[1, pl.ds(off, GB)], ebuf.at[q, 1],
                            isem.at[q]).wait()

    def drain(q, r):
      pltpu.make_async_copy(rows_v.at[r], agg_sh.at[ebuf.at[q, 1]],
                            ssem.at[r]).wait()
      if with_deg:
        pltpu.make_async_copy(ones_v, deg_sh.at[ebuf.at[q, 1]],
                              dsem.at[r]).wait()

    def gather_scatter(q, r):
      pltpu.async_copy(h_sh.at[ebuf.at[q, 0]], rows_v.at[r], gsem.at[r]).wait()
      pltpu.async_copy(rows_v.at[r], agg_sh.at[ebuf.at[q, 1]], ssem.at[r],
                       add=True)
      if with_deg:
        pltpu.async_copy(ones_v, deg_sh.at[ebuf.at[q, 1]], dsem.at[r],
                         add=True)

    pltpu.sync_copy(h_hbm.at[pl.ds(h0, HPT)], h_sh.at[pl.ds(h0, HPT)])
    pltpu.sync_copy(zeros_hbm.at[pl.ds(r0, RPT)], agg_sh.at[pl.ds(r0, RPT)])
    if with_deg:
      pltpu.sync_copy(zeros_hbm.at[pl.ds(r0, RPT), pl.ds(0, 16)],
                      deg_sh.at[pl.ds(r0, RPT)])
      pltpu.sync_copy(ones_hbm_ref(rest), ones_v)
    for q in range(4):
      idx_start(q, q)
    plsc.subcore_barrier()

    for q in range(4):
      idx_wait(q, q)
      if q >= 2:
        drain(q - 2, q % 2)
        idx_start(q + 2, q - 2)
      gather_scatter(q, q % 2)

    def g4body(g4, carry):
      for q in range(4):
        ph = g4 * 4 + q
        drain((q + 2) % 4, q % 2)
        pltpu.async_copy(e_hbm.at[0, pl.ds(e0 + (ph + 2) * GB, GB)],
                         ebuf.at[(q + 2) % 4, 0], isem.at[(q + 2) % 4])
        pltpu.async_copy(e_hbm.at[1, pl.ds(e0 + (ph + 2) * GB, GB)],
                         ebuf.at[(q + 2) % 4, 1], isem.at[(q + 2) % 4])
        idx_wait(ph, q)
        gather_scatter(q, q % 2)
      return carry

    lax.fori_loop(1, PH // 4, g4body, 0)

    for q in range(2):
      ph = PH - 2 + q
      drain((q + 2) % 4, q % 2)
      idx_wait(ph, q)
      gather_scatter(q, q % 2)
    for q in range(2):
      drain(q, q % 2)

    plsc.subcore_barrier()
    pltpu.sync_copy(agg_sh.at[pl.ds(r0, RPT)],
                    out_hbm.at[pl.ds(r0, RPT), pl.ds(cid * H, H)])
    if with_deg:
      pltpu.sync_copy(deg_sh.at[pl.ds(r0, RPT)],
                      deg_hbm.at[pl.ds(r0, RPT), pl.ds(cid * 16, 16)])

  def ones_hbm_ref(rest):
    raise AssertionError

  if with_deg:
    def body_deg(h_hbm, e_hbm, zeros_hbm, ones_hbm, out_hbm, deg_hbm,
                 ones_v, ebuf, rows_v, h_sh, agg_sh, deg_sh,
                 isem, gsem, ssem, dsem):
      nonlocal ones_hbm_ref
      ones_hbm_ref = lambda rest: ones_hbm
      return body(h_hbm, e_hbm, zeros_hbm, out_hbm, deg_hbm, ones_v, ebuf,
                  rows_v, h_sh, agg_sh, deg_sh, isem, gsem, ssem, dsem)

    out_type = (jax.ShapeDtypeStruct((NPAD, NC * H), jnp.float32),
                jax.ShapeDtypeStruct((NPAD, NC * 16), jnp.float32))
    scratch = [
        pltpu.VMEM((GB, 16), jnp.float32),
        pltpu.VMEM((4, 2, GB), jnp.int32),
        pltpu.VMEM((2, GB, H), jnp.float32),
        pltpu.VMEM_SHARED((N, H), jnp.float32),
        pltpu.VMEM_SHARED((NPAD, H), jnp.float32),
        pltpu.VMEM_SHARED((NPAD, 16), jnp.float32),
        pltpu.SemaphoreType.DMA((4,)),
        pltpu.SemaphoreType.DMA((2,)),
        pltpu.SemaphoreType.DMA((2,)),
        pltpu.SemaphoreType.DMA((2,)),
    ]
    fn = body_deg
  else:
    out_type = jax.ShapeDtypeStruct((NPAD, NC * H), jnp.float32)
    scratch = [
        pltpu.VMEM((4, 2, GB), jnp.int32),
        pltpu.VMEM((2, GB, H), jnp.float32),
        pltpu.VMEM_SHARED((N, H), jnp.float32),
        pltpu.VMEM_SHARED((NPAD, H), jnp.float32),
        pltpu.SemaphoreType.DMA((4,)),
        pltpu.SemaphoreType.DMA((2,)),
        pltpu.SemaphoreType.DMA((2,)),
    ]
    fn = body

  return pl.kernel(
      fn,
      out_type=out_type,
      mesh=_MESH,
      compiler_params=pltpu.CompilerParams(use_tc_tiling_on_sc=False),
      scratch_types=scratch,
  )


_sc_agg = _make_sc_agg(False)

def _sc_deg_body(e_hbm, ones_hbm, zeros_hbm, deg_hbm, ebuf_d, ones_v, deg_sh,
                 isem, dsem):
  cid = lax.axis_index("c")
  sid = lax.axis_index("s")
  wid = sid * NC + cid
  r0 = sid * RPT
  e0 = wid * E_W
  pltpu.sync_copy(zeros_hbm.at[pl.ds(r0, RPT), pl.ds(0, 16)],
                  deg_sh.at[pl.ds(r0, RPT)])
  pltpu.sync_copy(ones_hbm, ones_v)
  pltpu.sync_copy(e_hbm.at[1, pl.ds(e0, E_W)], ebuf_d)
  plsc.subcore_barrier()
  for ph in range(PH):
    pltpu.async_copy(ones_v, deg_sh.at[ebuf_d.at[pl.ds(ph * GB, GB)]], dsem,
                     add=True)
  for ph in range(PH):
    pltpu.make_async_copy(ones_v, deg_sh.at[ebuf_d.at[pl.ds(0, GB)]],
                          dsem).wait()
  plsc.subcore_barrier()
  pltpu.sync_copy(deg_sh.at[pl.ds(r0, RPT)],
                  deg_hbm.at[pl.ds(r0, RPT), pl.ds(cid * 16, 16)])


_sc_deg = pl.kernel(
    _sc_deg_body,
    out_type=jax.ShapeDtypeStruct((NPAD, NC * 16), jnp.float32),
    mesh=_MESH,
    compiler_params=pltpu.CompilerParams(use_tc_tiling_on_sc=False),
    scratch_types=[
        pltpu.VMEM((E_W,), jnp.int32),
        pltpu.VMEM((GB, 16), jnp.float32),
        pltpu.VMEM_SHARED((NPAD, 16), jnp.float32),
        pltpu.SemaphoreType.DMA,
        pltpu.SemaphoreType.DMA,
    ],
)


BN = 2000


def _fc_body(x_ref, w_ref, b_ref, o_ref):
  o_ref[...] = jnp.maximum(
      jnp.dot(x_ref[...], w_ref[...], preferred_element_type=jnp.float32)
      + b_ref[...], 0.0)


_fc = pl.pallas_call(
    _fc_body,
    grid=(N // BN,),
    in_specs=[
        pl.BlockSpec((BN, D_IN), lambda i: (i, 0)),
        pl.BlockSpec((D_IN, H), lambda i: (0, 0)),
        pl.BlockSpec((1, H), lambda i: (0, 0)),
    ],
    out_specs=pl.BlockSpec((BN, H), lambda i: (i, 0)),
    out_shape=jax.ShapeDtypeStruct((N, H), jnp.float32),
)


def _make_layer(with_ln):
  def body(parts_ref, degp_ref, h_ref, wn_ref, bn_ref, wr_ref, g_ref, be_ref,
           o_ref):
    p = parts_ref[...]
    s = p[:, :H] + p[:, H:]
    d = degp_ref[...]
    deg = jnp.maximum(d[:, 0:1] + d[:, 16:17], 1.0)
    hn = (jnp.dot(s / deg, wn_ref[...], preferred_element_type=jnp.float32)
          + bn_ref[...]
          + jnp.dot(h_ref[...], wr_ref[...], preferred_element_type=jnp.float32))
    if with_ln:
      mu = jnp.mean(hn, axis=-1, keepdims=True)
      var = jnp.mean((hn - mu) ** 2, axis=-1, keepdims=True)
      hn = g_ref[...] * (hn - mu) * lax.rsqrt(var + EPS) + be_ref[...]
      hn = jnp.maximum(hn, 0.0)
    o_ref[...] = hn

  return pl.pallas_call(
      body,
      grid=(N // BN,),
      in_specs=[
          pl.BlockSpec((BN, NC * H), lambda i: (i, 0)),
          pl.BlockSpec((BN, NC * 16), lambda i: (i, 0)),
          pl.BlockSpec((BN, H), lambda i: (i, 0)),
          pl.BlockSpec((H, H), lambda i: (0, 0)),
          pl.BlockSpec((1, H), lambda i: (0, 0)),
          pl.BlockSpec((H, H), lambda i: (0, 0)),
          pl.BlockSpec((1, H), lambda i: (0, 0)),
          pl.BlockSpec((1, H), lambda i: (0, 0)),
      ],
      out_specs=pl.BlockSpec((BN, H), lambda i: (i, 0)),
      out_shape=jax.ShapeDtypeStruct((N, H), jnp.float32),
  )


_layer_ln = _make_layer(True)
_layer_last = _make_layer(False)


def kernel(x, edge_index, batch, W_fc, b_fc, W_nbr, b_nbr, W_root, ln_g, ln_b):
  del batch
  zeros_h = jnp.zeros((NPAD, H), jnp.float32)
  ones_16 = jnp.ones((GB, 16), jnp.float32)

  h = _fc(x, W_fc, b_fc.reshape(1, H))
  degp = _sc_deg(edge_index, ones_16, zeros_h)
  for l in range(L):
    parts = _sc_agg(h, edge_index, zeros_h)
    f = _layer_ln if l < L - 1 else _layer_last
    g = ln_g[l] if l < L - 1 else ln_g[0]
    b = ln_b[l] if l < L - 1 else ln_b[0]
    h = f(parts, degp, h, W_nbr[l], b_nbr[l].reshape(1, H), W_root[l],
          g.reshape(1, H), b.reshape(1, H))
  return h


# speedup vs baseline: 16.6409x; 1.0293x over previous
import jax
import jax.numpy as jnp
from jax import lax
from jax.experimental import pallas as pl
from jax.experimental.pallas import tpu as pltpu
from jax.experimental.pallas import tpu_sc as plsc

N = 10000
E = 320000
D_IN = 128
H = 64
L = 3
EPS = 1e-5

NC = 2
NS = 16
NW = NC * NS
E_W = E // NW
GB = 200
PH = E_W // GB
NPAD = 10240
RPT = NPAD // NS
HPT = N // NS

_MESH = plsc.VectorSubcoreMesh(
    core_axis_name="c", subcore_axis_name="s", num_cores=NC, num_subcores=NS)


def _make_sc_agg(with_deg):
  def body(h_hbm, e_hbm, zeros_hbm, *rest):
    if with_deg:
      (out_hbm, deg_hbm, ones_v, ebuf, rows_v, h_sh, agg_sh, deg_sh,
       isem, gsem, ssem, dsem) = rest
    else:
      (out_hbm, ebuf, rows_v, h_sh, agg_sh, isem, gsem, ssem) = rest
    cid = lax.axis_index("c")
    sid = lax.axis_index("s")
    wid = sid * NC + cid
    r0 = sid * RPT
    h0 = sid * HPT
    e0 = wid * E_W

    def idx_start(ph, q):
      off = e0 + ph * GB
      pltpu.async_copy(e_hbm.at[0, pl.ds(off, GB)], ebuf.at[q, 0], isem.at[q])
      pltpu.async_copy(e_hbm.at[1, pl.ds(off, GB)], ebuf.at[q, 1], isem.at[q])

    def idx_wait(ph, q):
      off = e0 + ph * GB
      pltpu.make_async_copy(e_hbm.at[0, pl.ds(off, GB)], ebuf.at[q, 0],
                            isem.at[q]).wait()
      pltpu.make_async_copy(e_hbm.at[1, pl.ds(off, GB)], ebuf.at[q, 1],
                            isem.at[q]).wait()

    def drain(q, r):
      pltpu.make_async_copy(rows_v.at[r], agg_sh.at[ebuf.at[q, 1]],
                            ssem.at[r]).wait()
      if with_deg:
        pltpu.make_async_copy(ones_v, deg_sh.at[ebuf.at[q, 1]],
                              dsem.at[r]).wait()

    def gather_scatter(q, r):
      pltpu.async_copy(h_sh.at[ebuf.at[q, 0]], rows_v.at[r], gsem.at[r]).wait()
      pltpu.async_copy(rows_v.at[r], agg_sh.at[ebuf.at[q, 1]], ssem.at[r],
                       add=True)
      if with_deg:
        pltpu.async_copy(ones_v, deg_sh.at[ebuf.at[q, 1]], dsem.at[r],
                         add=True)

    pltpu.sync_copy(h_hbm.at[pl.ds(h0, HPT), pl.ds(0, H)],
                    h_sh.at[pl.ds(h0, HPT)])
    pltpu.sync_copy(zeros_hbm.at[pl.ds(r0, RPT)], agg_sh.at[pl.ds(r0, RPT)])
    if with_deg:
      pltpu.sync_copy(zeros_hbm.at[pl.ds(r0, RPT), pl.ds(0, 16)],
                      deg_sh.at[pl.ds(r0, RPT)])
      pltpu.sync_copy(ones_hbm_ref(rest), ones_v)
    for q in range(4):
      idx_start(q, q)
    plsc.subcore_barrier()

    for q in range(4):
      idx_wait(q, q)
      if q >= 2:
        drain(q - 2, q % 2)
        idx_start(q + 2, q - 2)
      gather_scatter(q, q % 2)

    def g4body(g4, carry):
      for q in range(4):
        ph = g4 * 4 + q
        drain((q + 2) % 4, q % 2)
        pltpu.async_copy(e_hbm.at[0, pl.ds(e0 + (ph + 2) * GB, GB)],
                         ebuf.at[(q + 2) % 4, 0], isem.at[(q + 2) % 4])
        pltpu.async_copy(e_hbm.at[1, pl.ds(e0 + (ph + 2) * GB, GB)],
                         ebuf.at[(q + 2) % 4, 1], isem.at[(q + 2) % 4])
        idx_wait(ph, q)
        gather_scatter(q, q % 2)
      return carry

    lax.fori_loop(1, PH // 4, g4body, 0)

    for q in range(2):
      ph = PH - 2 + q
      drain((q + 2) % 4, q % 2)
      idx_wait(ph, q)
      gather_scatter(q, q % 2)
    for q in range(2):
      drain(q, q % 2)

    plsc.subcore_barrier()
    pltpu.sync_copy(agg_sh.at[pl.ds(r0, RPT)],
                    out_hbm.at[pl.ds(r0, RPT), pl.ds(cid * H, H)])
    if with_deg:
      pltpu.sync_copy(deg_sh.at[pl.ds(r0, RPT)],
                      deg_hbm.at[pl.ds(r0, RPT), pl.ds(cid * 16, 16)])

  def ones_hbm_ref(rest):
    raise AssertionError

  if with_deg:
    def body_deg(h_hbm, e_hbm, zeros_hbm, ones_hbm, out_hbm, deg_hbm,
                 ones_v, ebuf, rows_v, h_sh, agg_sh, deg_sh,
                 isem, gsem, ssem, dsem):
      nonlocal ones_hbm_ref
      ones_hbm_ref = lambda rest: ones_hbm
      return body(h_hbm, e_hbm, zeros_hbm, out_hbm, deg_hbm, ones_v, ebuf,
                  rows_v, h_sh, agg_sh, deg_sh, isem, gsem, ssem, dsem)

    out_type = (jax.ShapeDtypeStruct((NPAD, NC * H), jnp.float32),
                jax.ShapeDtypeStruct((NPAD, NC * 16), jnp.float32))
    scratch = [
        pltpu.VMEM((GB, 16), jnp.float32),
        pltpu.VMEM((4, 2, GB), jnp.int32),
        pltpu.VMEM((2, GB, H), jnp.float32),
        pltpu.VMEM_SHARED((N, H), jnp.float32),
        pltpu.VMEM_SHARED((NPAD, H), jnp.float32),
        pltpu.VMEM_SHARED((NPAD, 16), jnp.float32),
        pltpu.SemaphoreType.DMA((4,)),
        pltpu.SemaphoreType.DMA((2,)),
        pltpu.SemaphoreType.DMA((2,)),
        pltpu.SemaphoreType.DMA((2,)),
    ]
    fn = body_deg
  else:
    out_type = jax.ShapeDtypeStruct((NPAD, NC * H), jnp.float32)
    scratch = [
        pltpu.VMEM((4, 2, GB), jnp.int32),
        pltpu.VMEM((2, GB, H), jnp.float32),
        pltpu.VMEM_SHARED((N, H), jnp.float32),
        pltpu.VMEM_SHARED((NPAD, H), jnp.float32),
        pltpu.SemaphoreType.DMA((4,)),
        pltpu.SemaphoreType.DMA((2,)),
        pltpu.SemaphoreType.DMA((2,)),
    ]
    fn = body

  return pl.kernel(
      fn,
      out_type=out_type,
      mesh=_MESH,
      compiler_params=pltpu.CompilerParams(use_tc_tiling_on_sc=False),
      scratch_types=scratch,
  )


_sc_agg = _make_sc_agg(False)

def _sc_deg_body(e_hbm, ones_hbm, zeros_hbm, deg_hbm, ebuf_d, ones_v, deg_sh,
                 isem, dsem):
  cid = lax.axis_index("c")
  sid = lax.axis_index("s")
  wid = sid * NC + cid
  r0 = sid * RPT
  e0 = wid * E_W
  pltpu.sync_copy(zeros_hbm.at[pl.ds(r0, RPT), pl.ds(0, 16)],
                  deg_sh.at[pl.ds(r0, RPT)])
  pltpu.sync_copy(ones_hbm, ones_v)
  pltpu.sync_copy(e_hbm.at[1, pl.ds(e0, E_W)], ebuf_d)
  plsc.subcore_barrier()
  for ph in range(PH):
    pltpu.async_copy(ones_v, deg_sh.at[ebuf_d.at[pl.ds(ph * GB, GB)]], dsem,
                     add=True)
  for ph in range(PH):
    pltpu.make_async_copy(ones_v, deg_sh.at[ebuf_d.at[pl.ds(0, GB)]],
                          dsem).wait()
  plsc.subcore_barrier()
  pltpu.sync_copy(deg_sh.at[pl.ds(r0, RPT)],
                  deg_hbm.at[pl.ds(r0, RPT), pl.ds(cid * 16, 16)])


_sc_deg = pl.kernel(
    _sc_deg_body,
    out_type=jax.ShapeDtypeStruct((NPAD, NC * 16), jnp.float32),
    mesh=_MESH,
    compiler_params=pltpu.CompilerParams(use_tc_tiling_on_sc=False),
    scratch_types=[
        pltpu.VMEM((E_W,), jnp.int32),
        pltpu.VMEM((GB, 16), jnp.float32),
        pltpu.VMEM_SHARED((NPAD, 16), jnp.float32),
        pltpu.SemaphoreType.DMA,
        pltpu.SemaphoreType.DMA,
    ],
)


BN = 2000


def _fc_body(x_ref, w_ref, b_ref, o_ref):
  hn = jnp.maximum(
      jnp.dot(x_ref[...], w_ref[...], preferred_element_type=jnp.float32)
      + b_ref[...], 0.0)
  o_ref[...] = jnp.concatenate([hn, jnp.zeros_like(hn)], axis=1)


_fc = pl.pallas_call(
    _fc_body,
    grid=(N // BN,),
    in_specs=[
        pl.BlockSpec((BN, D_IN), lambda i: (i, 0)),
        pl.BlockSpec((D_IN, H), lambda i: (0, 0)),
        pl.BlockSpec((1, H), lambda i: (0, 0)),
    ],
    out_specs=pl.BlockSpec((BN, 2 * H), lambda i: (i, 0)),
    out_shape=jax.ShapeDtypeStruct((N, 2 * H), jnp.float32),
)


def _make_layer(with_ln):
  def body(parts_ref, degp_ref, h_ref, wn_ref, bn_ref, wr_ref, g_ref, be_ref,
           o_ref):
    p = parts_ref[...]
    s = p[:, :H] + p[:, H:]
    d = degp_ref[...]
    deg = jnp.maximum(d[:, 0:1] + d[:, 16:17], 1.0)
    hn = (jnp.dot(s / deg, wn_ref[...], preferred_element_type=jnp.float32)
          + bn_ref[...]
          + jnp.dot(h_ref[:, :H], wr_ref[...],
                    preferred_element_type=jnp.float32))
    if with_ln:
      mu = jnp.mean(hn, axis=-1, keepdims=True)
      var = jnp.mean((hn - mu) ** 2, axis=-1, keepdims=True)
      hn = g_ref[...] * (hn - mu) * lax.rsqrt(var + EPS) + be_ref[...]
      hn = jnp.maximum(hn, 0.0)
    o_ref[...] = jnp.concatenate([hn, jnp.zeros_like(hn)], axis=1)

  return pl.pallas_call(
      body,
      grid=(N // BN,),
      in_specs=[
          pl.BlockSpec((BN, NC * H), lambda i: (i, 0)),
          pl.BlockSpec((BN, NC * 16), lambda i: (i, 0)),
          pl.BlockSpec((BN, 2 * H), lambda i: (i, 0)),
          pl.BlockSpec((H, H), lambda i: (0, 0)),
          pl.BlockSpec((1, H), lambda i: (0, 0)),
          pl.BlockSpec((H, H), lambda i: (0, 0)),
          pl.BlockSpec((1, H), lambda i: (0, 0)),
          pl.BlockSpec((1, H), lambda i: (0, 0)),
      ],
      out_specs=pl.BlockSpec((BN, 2 * H), lambda i: (i, 0)),
      out_shape=jax.ShapeDtypeStruct((N, 2 * H), jnp.float32),
  )


_layer_ln = _make_layer(True)
_layer_last = _make_layer(False)


def kernel(x, edge_index, batch, W_fc, b_fc, W_nbr, b_nbr, W_root, ln_g, ln_b):
  del batch
  zeros_h = jnp.zeros((NPAD, H), jnp.float32)
  ones_16 = jnp.ones((GB, 16), jnp.float32)

  h = _fc(x, W_fc, b_fc.reshape(1, H))
  degp = _sc_deg(edge_index, ones_16, zeros_h)
  for l in range(L):
    parts = _sc_agg(h, edge_index, zeros_h)
    f = _layer_ln if l < L - 1 else _layer_last
    g = ln_g[l] if l < L - 1 else ln_g[0]
    b = ln_b[l] if l < L - 1 else ln_b[0]
    h = f(parts, degp, h, W_nbr[l], b_nbr[l].reshape(1, H), W_root[l],
          g.reshape(1, H), b.reshape(1, H))
  return h[:, :H]
